# Initial kernel scaffold; baseline (speedup 1.0000x reference)
import functools

import jax
import jax.numpy as jnp
from jax import lax
from jax.experimental import pallas as pl
from jax.experimental.pallas import tpu as pltpu
from jax.experimental.pallas import tpu_sc as plsc

NC = 2
NS = 16
NW = NC * NS

DEG_W = 16


def _pick_chunk(e_per_tile):
    for c in (128, 120, 112, 104, 96, 88, 80, 72, 64, 56, 48, 40, 32, 24, 16, 8):
        if e_per_tile % c == 0:
            return c
    raise ValueError(f"edge count per tile {e_per_tile} not divisible by 8")


def _pick_rows_chunk(rows_per_tile):
    for c in range(128, 0, -1):
        if rows_per_tile % c == 0:
            return c
    raise ValueError("bad row count")


@functools.lru_cache(maxsize=None)
def _make_deg_kernel(E, N):
    e_per_tile = E // NW
    C = _pick_chunk(e_per_tile)
    n_chunks = e_per_tile // C
    rpt = N // NS
    RB = _pick_rows_chunk(rpt)
    n_rb = rpt // RB
    mesh = plsc.VectorSubcoreMesh(core_axis_name="c", subcore_axis_name="s")

    @functools.partial(
        pl.kernel,
        out_type=jax.ShapeDtypeStruct((NC, N, DEG_W), jnp.float32),
        mesh=mesh,
        scratch_types=[
            pltpu.VMEM((C,), jnp.int32),
            pltpu.VMEM((C, DEG_W), jnp.float32),
            pltpu.VMEM((RB, DEG_W), jnp.float32),
            pltpu.VMEM_SHARED((N, DEG_W), jnp.float32),
        ],
    )
    def deg_kernel(dst_hbm, out_hbm, dst_v, ones_v, bounce_v, acc):
        c = lax.axis_index("c")
        s = lax.axis_index("s")
        wid = c * NS + s

        def fill_ones(i, _):
            ones_v[i, :] = jnp.ones((DEG_W,), jnp.float32)
            return 0

        lax.fori_loop(0, C, fill_ones, 0)

        def fill_zero(i, _):
            bounce_v[i, :] = jnp.zeros((DEG_W,), jnp.float32)
            return 0

        lax.fori_loop(0, RB, fill_zero, 0)

        def zero_chunk(j, _):
            pltpu.sync_copy(bounce_v, acc.at[pl.ds(s * rpt + j * RB, RB)])
            return 0

        lax.fori_loop(0, n_rb, zero_chunk, 0)
        plsc.subcore_barrier()

        base0 = wid * e_per_tile

        def chunk(i, _):
            pltpu.sync_copy(dst_hbm.at[pl.ds(base0 + i * C, C)], dst_v)
            pltpu.sync_copy(ones_v, acc.at[dst_v], add=True)
            return 0

        lax.fori_loop(0, n_chunks, chunk, 0)
        plsc.subcore_barrier()

        def out_chunk(j, _):
            pltpu.sync_copy(acc.at[pl.ds(s * rpt + j * RB, RB)], bounce_v)
            pltpu.sync_copy(bounce_v, out_hbm.at[c, pl.ds(s * rpt + j * RB, RB)])
            return 0

        lax.fori_loop(0, n_rb, out_chunk, 0)

    return deg_kernel


@functools.lru_cache(maxsize=None)
def _make_scatter_kernel(E, N, D):
    e_per_tile = E // NW
    C = _pick_chunk(e_per_tile)
    n_chunks = e_per_tile // C
    rpt = N // NS
    RB = _pick_rows_chunk(rpt)
    n_rb = rpt // RB
    mesh = plsc.VectorSubcoreMesh(core_axis_name="c", subcore_axis_name="s")

    @functools.partial(
        pl.kernel,
        out_type=jax.ShapeDtypeStruct((NC, N, D), jnp.float32),
        mesh=mesh,
        scratch_types=[
            pltpu.VMEM((C,), jnp.int32),
            pltpu.VMEM((C,), jnp.int32),
            pltpu.VMEM((C, D), jnp.float32),
            pltpu.VMEM((RB, D), jnp.float32),
            pltpu.VMEM_SHARED((N, D), jnp.float32),
            pltpu.SemaphoreType.DMA,
        ],
    )
    def scatter_kernel(g_hbm, src_hbm, dst_hbm, out_hbm,
                       src_v, dst_v, rows_v, bounce_v, acc, sem):
        c = lax.axis_index("c")
        s = lax.axis_index("s")
        wid = c * NS + s

        def fill_zero(i, _):
            def fz(j, _):
                bounce_v[i, pl.ds(j * 16, 16)] = jnp.zeros((16,), jnp.float32)
                return 0
            lax.fori_loop(0, D // 16, fz, 0)
            return 0

        lax.fori_loop(0, RB, fill_zero, 0)

        def zero_chunk(j, _):
            pltpu.sync_copy(bounce_v, acc.at[pl.ds(s * rpt + j * RB, RB)])
            return 0

        lax.fori_loop(0, n_rb, zero_chunk, 0)
        plsc.subcore_barrier()

        base0 = wid * e_per_tile

        def chunk(i, _):
            base = base0 + i * C
            pltpu.sync_copy(src_hbm.at[pl.ds(base, C)], src_v)
            pltpu.sync_copy(dst_hbm.at[pl.ds(base, C)], dst_v)
            pltpu.async_copy(g_hbm.at[src_v], rows_v, sem).wait()
            pltpu.sync_copy(rows_v, acc.at[dst_v], add=True)
            return 0

        lax.fori_loop(0, n_chunks, chunk, 0)
        plsc.subcore_barrier()

        def out_chunk(j, _):
            pltpu.sync_copy(acc.at[pl.ds(s * rpt + j * RB, RB)], bounce_v)
            pltpu.sync_copy(bounce_v, out_hbm.at[c, pl.ds(s * rpt + j * RB, RB)])
            return 0

        lax.fori_loop(0, n_rb, out_chunk, 0)

    return scatter_kernel


_R = 2500


def _mm_body(x_ref, w_ref, o_ref):
    o_ref[...] = jnp.dot(x_ref[...], w_ref[...],
                         preferred_element_type=jnp.float32)


def _tc_mm(x, w):
    n, d = x.shape
    r = _R if n % _R == 0 else 8
    return pl.pallas_call(
        _mm_body,
        grid=(n // r,),
        in_specs=[
            pl.BlockSpec((r, d), lambda i: (i, 0)),
            pl.BlockSpec((d, w.shape[1]), lambda i: (0, 0)),
        ],
        out_specs=pl.BlockSpec((r, w.shape[1]), lambda i: (i, 0)),
        out_shape=jax.ShapeDtypeStruct((n, w.shape[1]), jnp.float32),
    )(x, w)


def _dinv_from_deg(deg_ref):
    deg = deg_ref[0, :, 0:1] + deg_ref[1, :, 0:1] + 1.0
    return lax.rsqrt(deg)


def _scale_body(t_ref, deg_ref, o_ref):
    o_ref[...] = t_ref[...] * _dinv_from_deg(deg_ref)


def _tc_scale(t, degp):
    n, d = t.shape
    r = _R if n % _R == 0 else 8
    return pl.pallas_call(
        _scale_body,
        grid=(n // r,),
        in_specs=[
            pl.BlockSpec((r, d), lambda i: (i, 0)),
            pl.BlockSpec((NC, r, DEG_W), lambda i: (0, i, 0)),
        ],
        out_specs=pl.BlockSpec((r, d), lambda i: (i, 0)),
        out_shape=jax.ShapeDtypeStruct((n, d), jnp.float32),
    )(t, degp)


def _layer_body(s_ref, g_ref, deg_ref, b_ref, w_ref, o_ref):
    dinv = _dinv_from_deg(deg_ref)
    tot = s_ref[0] + s_ref[1] + g_ref[...]
    h = jnp.maximum(dinv * tot + b_ref[...], 0.0)
    o_ref[...] = jnp.dot(h, w_ref[...],
                         preferred_element_type=jnp.float32) * dinv


def _tc_layer(S, g, degp, b, w):
    n, d = g.shape
    r = _R if n % _R == 0 else 8
    return pl.pallas_call(
        _layer_body,
        grid=(n // r,),
        in_specs=[
            pl.BlockSpec((NC, r, d), lambda i: (0, i, 0)),
            pl.BlockSpec((r, d), lambda i: (i, 0)),
            pl.BlockSpec((NC, r, DEG_W), lambda i: (0, i, 0)),
            pl.BlockSpec((1, d), lambda i: (0, 0)),
            pl.BlockSpec((d, d), lambda i: (0, 0)),
        ],
        out_specs=pl.BlockSpec((r, d), lambda i: (i, 0)),
        out_shape=jax.ShapeDtypeStruct((n, d), jnp.float32),
    )(S, g, degp, b, w)


def _final_body(s_ref, g_ref, deg_ref, b_ref, w_ref, b3_ref, o_ref):
    dinv = _dinv_from_deg(deg_ref)
    tot = s_ref[0] + s_ref[1] + g_ref[...]
    h = jnp.maximum(dinv * tot + b_ref[...], 0.0)
    o_ref[...] = jnp.dot(h, w_ref[...],
                         preferred_element_type=jnp.float32) + b3_ref[...]


def _tc_final(S, g, degp, b, w_pad, b3_pad):
    n, d = g.shape
    r = _R if n % _R == 0 else 8
    dp = w_pad.shape[1]
    return pl.pallas_call(
        _final_body,
        grid=(n // r,),
        in_specs=[
            pl.BlockSpec((NC, r, d), lambda i: (0, i, 0)),
            pl.BlockSpec((r, d), lambda i: (i, 0)),
            pl.BlockSpec((NC, r, DEG_W), lambda i: (0, i, 0)),
            pl.BlockSpec((1, d), lambda i: (0, 0)),
            pl.BlockSpec((d, dp), lambda i: (0, 0)),
            pl.BlockSpec((1, dp), lambda i: (0, 0)),
        ],
        out_specs=pl.BlockSpec((r, dp), lambda i: (i, 0)),
        out_shape=jax.ShapeDtypeStruct((n, dp), jnp.float32),
    )(S, g, degp, b, w_pad, b3_pad)


def kernel(x, edge_index, W1, b1, W2, b2, W3, b3):
    n, d = x.shape
    e = edge_index.shape[1]
    src = edge_index[0].astype(jnp.int32)
    dst = edge_index[1].astype(jnp.int32)

    n_classes = W3.shape[1]
    w3_pad = jnp.pad(W3.astype(jnp.float32), ((0, 0), (0, d - n_classes)))
    b3_pad = jnp.pad(b3.astype(jnp.float32), (0, d - n_classes)).reshape(1, d)
    b1r = b1.reshape(1, d)
    b2r = b2.reshape(1, d)

    deg_fn = _make_deg_kernel(e, n)
    scat_fn = _make_scatter_kernel(e, n, d)

    degp = deg_fn(dst)
    t1 = _tc_mm(x, W1)
    g1 = _tc_scale(t1, degp)
    S1 = scat_fn(g1, src, dst)
    g2 = _tc_layer(S1, g1, degp, b1r, W2)
    S2 = scat_fn(g2, src, dst)
    out = _tc_final(S2, g2, degp, b2r, w3_pad, b3_pad)
    return out[:, :n_classes]


# speedup vs baseline: 12.1033x; 12.1033x over previous
import functools

import jax
import jax.numpy as jnp
from jax import lax
from jax.experimental import pallas as pl
from jax.experimental.pallas import tpu as pltpu
from jax.experimental.pallas import tpu_sc as plsc

NC = 2
NS = 16
NW = NC * NS

DEG_W = 128


def _pick_chunk(e_per_tile):
    for c in (128, 120, 112, 104, 96, 88, 80, 72, 64, 56, 48, 40, 32, 24, 16, 8):
        if e_per_tile % c == 0:
            return c
    raise ValueError(f"edge count per tile {e_per_tile} not divisible by 8")


def _pick_rows_chunk(rows_per_tile):
    for c in range(128, 0, -1):
        if rows_per_tile % c == 0 and c % 8 == 0:
            return c
    raise ValueError("bad row count")


def _pad_rows(N):
    q = NS * 8
    return ((N + q - 1) // q) * q


@functools.lru_cache(maxsize=None)
def _make_deg_kernel(E, N):
    e_per_tile = E // NW
    C = _pick_chunk(e_per_tile)
    n_chunks = e_per_tile // C
    Np = _pad_rows(N)
    rpt = Np // NS
    RB = _pick_rows_chunk(rpt)
    n_rb = rpt // RB
    mesh = plsc.VectorSubcoreMesh(core_axis_name="c", subcore_axis_name="s", num_cores=NC, num_subcores=NS)

    @functools.partial(
        pl.kernel,
        out_type=jax.ShapeDtypeStruct((NC, Np, DEG_W), jnp.float32),
        mesh=mesh,
        scratch_types=[
            pltpu.VMEM((C,), jnp.int32),
            pltpu.VMEM((C, DEG_W), jnp.float32),
            pltpu.VMEM((RB, DEG_W), jnp.float32),
            pltpu.VMEM_SHARED((Np, DEG_W), jnp.float32),
        ],
    )
    def deg_kernel(dst_hbm, out_hbm, dst_v, ones_v, bounce_v, acc):
        c = lax.axis_index("c")
        s = lax.axis_index("s")
        wid = c * NS + s

        def fill_ones(i, _):
            def fo(j, _):
                ones_v[i, pl.ds(j * 16, 16)] = jnp.ones((16,), jnp.float32)
                return 0
            lax.fori_loop(0, DEG_W // 16, fo, 0)
            return 0

        lax.fori_loop(0, C, fill_ones, 0)

        def fill_zero(i, _):
            def fz(j, _):
                bounce_v[i, pl.ds(j * 16, 16)] = jnp.zeros((16,), jnp.float32)
                return 0
            lax.fori_loop(0, DEG_W // 16, fz, 0)
            return 0

        lax.fori_loop(0, RB, fill_zero, 0)

        def zero_chunk(j, _):
            pltpu.sync_copy(bounce_v, acc.at[pl.ds(s * rpt + j * RB, RB)])
            return 0

        lax.fori_loop(0, n_rb, zero_chunk, 0)
        plsc.subcore_barrier()

        base0 = wid * e_per_tile

        def chunk(i, _):
            pltpu.sync_copy(dst_hbm.at[pl.ds(base0 + i * C, C)], dst_v)
            pltpu.sync_copy(ones_v, acc.at[dst_v], add=True)
            return 0

        lax.fori_loop(0, n_chunks, chunk, 0)
        plsc.subcore_barrier()

        def out_chunk(j, _):
            pltpu.sync_copy(acc.at[pl.ds(s * rpt + j * RB, RB)], bounce_v)
            pltpu.sync_copy(bounce_v, out_hbm.at[c, pl.ds(s * rpt + j * RB, RB)])
            return 0

        lax.fori_loop(0, n_rb, out_chunk, 0)

    return deg_kernel


@functools.lru_cache(maxsize=None)
def _make_scatter_kernel(E, N, D):
    e_per_tile = E // NW
    C = _pick_chunk(e_per_tile)
    n_chunks = e_per_tile // C
    Np = _pad_rows(N)
    rpt = Np // NS
    RB = _pick_rows_chunk(rpt)
    n_rb = rpt // RB
    mesh = plsc.VectorSubcoreMesh(core_axis_name="c", subcore_axis_name="s", num_cores=NC, num_subcores=NS)

    @functools.partial(
        pl.kernel,
        out_type=jax.ShapeDtypeStruct((NC, Np, D), jnp.float32),
        mesh=mesh,
        scratch_types=[
            pltpu.VMEM((C,), jnp.int32),
            pltpu.VMEM((C,), jnp.int32),
            pltpu.VMEM((C, D), jnp.float32),
            pltpu.VMEM((RB, D), jnp.float32),
            pltpu.VMEM_SHARED((Np, D), jnp.float32),
            pltpu.SemaphoreType.DMA,
        ],
    )
    def scatter_kernel(g_hbm, src_hbm, dst_hbm, out_hbm,
                       src_v, dst_v, rows_v, bounce_v, acc, sem):
        c = lax.axis_index("c")
        s = lax.axis_index("s")
        wid = c * NS + s

        def fill_zero(i, _):
            def fz(j, _):
                bounce_v[i, pl.ds(j * 16, 16)] = jnp.zeros((16,), jnp.float32)
                return 0
            lax.fori_loop(0, D // 16, fz, 0)
            return 0

        lax.fori_loop(0, RB, fill_zero, 0)

        def zero_chunk(j, _):
            pltpu.sync_copy(bounce_v, acc.at[pl.ds(s * rpt + j * RB, RB)])
            return 0

        lax.fori_loop(0, n_rb, zero_chunk, 0)
        plsc.subcore_barrier()

        base0 = wid * e_per_tile

        def chunk(i, _):
            base = base0 + i * C
            pltpu.sync_copy(src_hbm.at[pl.ds(base, C)], src_v)
            pltpu.sync_copy(dst_hbm.at[pl.ds(base, C)], dst_v)
            pltpu.async_copy(g_hbm.at[src_v], rows_v, sem).wait()
            pltpu.sync_copy(rows_v, acc.at[dst_v], add=True)
            return 0

        lax.fori_loop(0, n_chunks, chunk, 0)
        plsc.subcore_barrier()

        def out_chunk(j, _):
            pltpu.sync_copy(acc.at[pl.ds(s * rpt + j * RB, RB)], bounce_v)
            pltpu.sync_copy(bounce_v, out_hbm.at[c, pl.ds(s * rpt + j * RB, RB)])
            return 0

        lax.fori_loop(0, n_rb, out_chunk, 0)

    return scatter_kernel


_R = 2000


def _mm_body(x_ref, w_ref, o_ref):
    o_ref[...] = jnp.dot(x_ref[...], w_ref[...],
                         preferred_element_type=jnp.float32)


def _tc_mm(x, w):
    n, d = x.shape
    r = _R if n % _R == 0 else 8
    return pl.pallas_call(
        _mm_body,
        grid=(n // r,),
        in_specs=[
            pl.BlockSpec((r, d), lambda i: (i, 0)),
            pl.BlockSpec((d, w.shape[1]), lambda i: (0, 0)),
        ],
        out_specs=pl.BlockSpec((r, w.shape[1]), lambda i: (i, 0)),
        out_shape=jax.ShapeDtypeStruct((n, w.shape[1]), jnp.float32),
    )(x, w)


def _dinv_from_deg(deg_ref):
    deg = deg_ref[0, :, 0:1] + deg_ref[1, :, 0:1] + 1.0
    return lax.rsqrt(deg)


def _scale_body(t_ref, deg_ref, o_ref):
    o_ref[...] = t_ref[...] * _dinv_from_deg(deg_ref)


def _tc_scale(t, degp):
    n, d = t.shape
    r = _R if n % _R == 0 else 8
    return pl.pallas_call(
        _scale_body,
        grid=(n // r,),
        in_specs=[
            pl.BlockSpec((r, d), lambda i: (i, 0)),
            pl.BlockSpec((NC, r, DEG_W), lambda i: (0, i, 0)),
        ],
        out_specs=pl.BlockSpec((r, d), lambda i: (i, 0)),
        out_shape=jax.ShapeDtypeStruct((n, d), jnp.float32),
    )(t, degp)


def _layer_body(s_ref, g_ref, deg_ref, b_ref, w_ref, o_ref):
    dinv = _dinv_from_deg(deg_ref)
    tot = s_ref[0] + s_ref[1] + g_ref[...]
    h = jnp.maximum(dinv * tot + b_ref[...], 0.0)
    o_ref[...] = jnp.dot(h, w_ref[...],
                         preferred_element_type=jnp.float32) * dinv


def _tc_layer(S, g, degp, b, w):
    n, d = g.shape
    r = _R if n % _R == 0 else 8
    return pl.pallas_call(
        _layer_body,
        grid=(n // r,),
        in_specs=[
            pl.BlockSpec((NC, r, d), lambda i: (0, i, 0)),
            pl.BlockSpec((r, d), lambda i: (i, 0)),
            pl.BlockSpec((NC, r, DEG_W), lambda i: (0, i, 0)),
            pl.BlockSpec((1, d), lambda i: (0, 0)),
            pl.BlockSpec((d, d), lambda i: (0, 0)),
        ],
        out_specs=pl.BlockSpec((r, d), lambda i: (i, 0)),
        out_shape=jax.ShapeDtypeStruct((n, d), jnp.float32),
    )(S, g, degp, b, w)


def _final_body(s_ref, g_ref, deg_ref, b_ref, w_ref, b3_ref, o_ref):
    dinv = _dinv_from_deg(deg_ref)
    tot = s_ref[0] + s_ref[1] + g_ref[...]
    h = jnp.maximum(dinv * tot + b_ref[...], 0.0)
    o_ref[...] = jnp.dot(h, w_ref[...],
                         preferred_element_type=jnp.float32) + b3_ref[...]


def _tc_final(S, g, degp, b, w_pad, b3_pad):
    n, d = g.shape
    r = _R if n % _R == 0 else 8
    dp = w_pad.shape[1]
    return pl.pallas_call(
        _final_body,
        grid=(n // r,),
        in_specs=[
            pl.BlockSpec((NC, r, d), lambda i: (0, i, 0)),
            pl.BlockSpec((r, d), lambda i: (i, 0)),
            pl.BlockSpec((NC, r, DEG_W), lambda i: (0, i, 0)),
            pl.BlockSpec((1, d), lambda i: (0, 0)),
            pl.BlockSpec((d, dp), lambda i: (0, 0)),
            pl.BlockSpec((1, dp), lambda i: (0, 0)),
        ],
        out_specs=pl.BlockSpec((r, dp), lambda i: (i, 0)),
        out_shape=jax.ShapeDtypeStruct((n, dp), jnp.float32),
    )(S, g, degp, b, w_pad, b3_pad)


def kernel(x, edge_index, W1, b1, W2, b2, W3, b3):
    n, d = x.shape
    e = edge_index.shape[1]
    src = edge_index[0].astype(jnp.int32)
    dst = edge_index[1].astype(jnp.int32)

    n_classes = W3.shape[1]
    w3_pad = jnp.pad(W3.astype(jnp.float32), ((0, 0), (0, d - n_classes)))
    b3_pad = jnp.pad(b3.astype(jnp.float32), (0, d - n_classes)).reshape(1, d)
    b1r = b1.reshape(1, d)
    b2r = b2.reshape(1, d)

    deg_fn = _make_deg_kernel(e, n)
    scat_fn = _make_scatter_kernel(e, n, d)

    degp = deg_fn(dst)
    t1 = _tc_mm(x, W1)
    g1 = _tc_scale(t1, degp)
    S1 = scat_fn(g1, src, dst)
    g2 = _tc_layer(S1, g1, degp, b1r, W2)
    S2 = scat_fn(g2, src, dst)
    out = _tc_final(S2, g2, degp, b2r, w3_pad, b3_pad)
    return out[:, :n_classes]
